# baseline (speedup 1.0000x reference)
import functools

import jax
import jax.numpy as jnp
from jax import lax
from jax.experimental import pallas as pl
from jax.experimental.pallas import tpu as pltpu
from jax.experimental.pallas import tpu_sc as plsc

_NUM_CORES = 2
_NUM_SUBCORES = 16
_NUM_WORKERS = _NUM_CORES * _NUM_SUBCORES


def kernel(inputs, pos_table):
    seq_len = inputs.shape[1]
    table_len, embed_dim = pos_table.shape
    assert seq_len % _NUM_WORKERS == 0
    rows_per_worker = seq_len // _NUM_WORKERS

    mesh = plsc.VectorSubcoreMesh(
        core_axis_name="c", subcore_axis_name="s"
    )

    @functools.partial(
        pl.kernel,
        mesh=mesh,
        out_type=jax.ShapeDtypeStruct((seq_len, embed_dim), jnp.float32),
    )
    def copy_rows(table_hbm, out_hbm):
        wid = lax.axis_index("s") * _NUM_CORES + lax.axis_index("c")
        base = wid * rows_per_worker
        pltpu.sync_copy(
            table_hbm.at[pl.ds(base, rows_per_worker)],
            out_hbm.at[pl.ds(base, rows_per_worker)],
        )

    return copy_rows(pos_table)


# speedup vs baseline: 24.6130x; 24.6130x over previous
import functools

import jax
import jax.numpy as jnp
from jax import lax
from jax.experimental import pallas as pl
from jax.experimental.pallas import tpu as pltpu
from jax.experimental.pallas import tpu_sc as plsc

_NUM_CORES = 2
_NUM_SUBCORES = 16
_NUM_WORKERS = _NUM_CORES * _NUM_SUBCORES
_CHUNK_ROWS = 16
_NBUF = 4


def kernel(inputs, pos_table):
    seq_len = inputs.shape[1]
    table_len, embed_dim = pos_table.shape
    assert seq_len % (_NUM_WORKERS * _CHUNK_ROWS) == 0
    rows_per_worker = seq_len // _NUM_WORKERS
    nchunks = rows_per_worker // _CHUNK_ROWS

    mesh = plsc.VectorSubcoreMesh(core_axis_name="c", subcore_axis_name="s")

    @functools.partial(
        pl.kernel,
        mesh=mesh,
        out_type=jax.ShapeDtypeStruct((seq_len, embed_dim), jnp.float32),
        scratch_types=[
            pltpu.VMEM((_NBUF, _CHUNK_ROWS, embed_dim), jnp.float32),
            pltpu.SemaphoreType.DMA,
            pltpu.SemaphoreType.DMA,
        ],
    )
    def copy_rows(table_hbm, out_hbm, buf, sem_in, sem_out):
        wid = lax.axis_index("s") * _NUM_CORES + lax.axis_index("c")
        base = wid * rows_per_worker

        def start_in(i):
            return pltpu.async_copy(
                table_hbm.at[pl.ds(base + i * _CHUNK_ROWS, _CHUNK_ROWS)],
                buf.at[i % _NBUF],
                sem_in,
            )

        def start_out(i):
            return pltpu.async_copy(
                buf.at[i % _NBUF],
                out_hbm.at[pl.ds(base + i * _CHUNK_ROWS, _CHUNK_ROWS)],
                sem_out,
            )

        ins = [None] * nchunks
        outs = [None] * nchunks
        for i in range(min(_NBUF, nchunks)):
            ins[i] = start_in(i)
        for i in range(nchunks):
            ins[i].wait()
            outs[i] = start_out(i)
            nxt = i + _NBUF
            if nxt < nchunks:
                outs[i].wait()
                ins[nxt] = start_in(nxt)
        for i in range(max(nchunks - _NBUF, 0), nchunks):
            outs[i].wait()

    return copy_rows(pos_table)


# speedup vs baseline: 24.9528x; 1.0138x over previous
import functools

import jax
import jax.numpy as jnp
from jax import lax
from jax.experimental import pallas as pl
from jax.experimental.pallas import tpu as pltpu
from jax.experimental.pallas import tpu_sc as plsc

_NUM_CORES = 2
_NUM_SUBCORES = 16
_NUM_WORKERS = _NUM_CORES * _NUM_SUBCORES
_CHUNK_ROWS = 32
_NBUF = 3


def kernel(inputs, pos_table):
    seq_len = inputs.shape[1]
    table_len, embed_dim = pos_table.shape
    assert seq_len % (_NUM_WORKERS * _CHUNK_ROWS) == 0
    rows_per_worker = seq_len // _NUM_WORKERS
    nchunks = rows_per_worker // _CHUNK_ROWS

    mesh = plsc.VectorSubcoreMesh(core_axis_name="c", subcore_axis_name="s")

    @functools.partial(
        pl.kernel,
        mesh=mesh,
        out_type=jax.ShapeDtypeStruct((seq_len, embed_dim), jnp.float32),
        scratch_types=[
            pltpu.VMEM((_NBUF, _CHUNK_ROWS, embed_dim), jnp.float32),
            pltpu.SemaphoreType.DMA,
            pltpu.SemaphoreType.DMA,
        ],
    )
    def copy_rows(table_hbm, out_hbm, buf, sem_in, sem_out):
        wid = lax.axis_index("s") * _NUM_CORES + lax.axis_index("c")
        base = wid * rows_per_worker

        def start_in(i):
            return pltpu.async_copy(
                table_hbm.at[pl.ds(base + i * _CHUNK_ROWS, _CHUNK_ROWS)],
                buf.at[i % _NBUF],
                sem_in,
            )

        def start_out(i):
            return pltpu.async_copy(
                buf.at[i % _NBUF],
                out_hbm.at[pl.ds(base + i * _CHUNK_ROWS, _CHUNK_ROWS)],
                sem_out,
            )

        ins = [None] * nchunks
        outs = [None] * nchunks
        for i in range(min(_NBUF, nchunks)):
            ins[i] = start_in(i)
        for i in range(nchunks):
            ins[i].wait()
            outs[i] = start_out(i)
            nxt = i + _NBUF
            if nxt < nchunks:
                outs[i].wait()
                ins[nxt] = start_in(nxt)
        for i in range(max(nchunks - _NBUF, 0), nchunks):
            outs[i].wait()

    return copy_rows(pos_table)


# speedup vs baseline: 25.8499x; 1.0359x over previous
import functools

import jax
import jax.numpy as jnp
from jax import lax
from jax.experimental import pallas as pl
from jax.experimental.pallas import tpu as pltpu
from jax.experimental.pallas import tpu_sc as plsc

_NUM_CORES = 2
_NUM_SUBCORES = 16
_NUM_WORKERS = _NUM_CORES * _NUM_SUBCORES
_A_CHUNK = 48
_A_NBUF = 2
_A_ROWS = 192
_B_CHUNK = 16
_B_NBUF = 2
_B_ROWS = 64


def kernel(inputs, pos_table):
    seq_len = inputs.shape[1]
    table_len, embed_dim = pos_table.shape
    rows_per_worker = seq_len // _NUM_WORKERS
    assert _A_ROWS + _B_ROWS == rows_per_worker
    na = _A_ROWS // _A_CHUNK
    nb = _B_ROWS // _B_CHUNK

    mesh = plsc.VectorSubcoreMesh(core_axis_name="c", subcore_axis_name="s")

    @functools.partial(
        pl.kernel,
        mesh=mesh,
        out_type=jax.ShapeDtypeStruct((seq_len, embed_dim), jnp.float32),
        scratch_types=[
            pltpu.VMEM((_A_NBUF, _A_CHUNK, embed_dim), jnp.float32),
            pltpu.VMEM_SHARED(
                (_NUM_SUBCORES, _B_NBUF, _B_CHUNK, embed_dim), jnp.float32
            ),
            pltpu.SemaphoreType.DMA,
            pltpu.SemaphoreType.DMA,
            pltpu.SemaphoreType.DMA,
            pltpu.SemaphoreType.DMA,
        ],
    )
    def copy_rows(table_hbm, out_hbm, tbuf, sbuf, sem_ta, sem_tb, sem_sa, sem_sb):
        sid = lax.axis_index("s")
        wid = sid * _NUM_CORES + lax.axis_index("c")
        base_a = wid * rows_per_worker
        base_b = base_a + _A_ROWS

        def a_in(i):
            return pltpu.async_copy(
                table_hbm.at[pl.ds(base_a + i * _A_CHUNK, _A_CHUNK)],
                tbuf.at[i % _A_NBUF],
                sem_ta,
            )

        def a_out(i):
            return pltpu.async_copy(
                tbuf.at[i % _A_NBUF],
                out_hbm.at[pl.ds(base_a + i * _A_CHUNK, _A_CHUNK)],
                sem_tb,
            )

        def b_in(i):
            return pltpu.async_copy(
                table_hbm.at[pl.ds(base_b + i * _B_CHUNK, _B_CHUNK)],
                sbuf.at[sid, i % _B_NBUF],
                sem_sa,
            )

        def b_out(i):
            return pltpu.async_copy(
                sbuf.at[sid, i % _B_NBUF],
                out_hbm.at[pl.ds(base_b + i * _B_CHUNK, _B_CHUNK)],
                sem_sb,
            )

        a_ins = [None] * na
        a_outs = [None] * na
        b_ins = [None] * nb
        b_outs = [None] * nb
        for i in range(min(_A_NBUF, na)):
            a_ins[i] = a_in(i)
        for i in range(min(_B_NBUF, nb)):
            b_ins[i] = b_in(i)
        for i in range(max(na, nb)):
            if i < nb:
                b_ins[i].wait()
                b_outs[i] = b_out(i)
            if i < na:
                a_ins[i].wait()
                a_outs[i] = a_out(i)
            if i + _B_NBUF < nb:
                b_outs[i].wait()
                b_ins[i + _B_NBUF] = b_in(i + _B_NBUF)
            if i + _A_NBUF < na:
                a_outs[i].wait()
                a_ins[i + _A_NBUF] = a_in(i + _A_NBUF)
        for i in range(max(na - _A_NBUF, 0), na):
            a_outs[i].wait()
        for i in range(max(nb - _B_NBUF, 0), nb):
            b_outs[i].wait()

    return copy_rows(pos_table)
